# baseline (speedup 1.0000x reference)
import functools

import jax
import jax.numpy as jnp
from jax import lax
from jax.experimental import pallas as pl
from jax.experimental.pallas import tpu as pltpu
from jax.experimental.pallas import tpu_sc as plsc

N = 10000
F = 128
E = 320000
ALPHA = 0.3

NSUB = 16
ROWS_MAIN = 632
ROWS_LAST = N - 15 * ROWS_MAIN
CHUNK = 128
EDGES_PER_SUB = E // NSUB
NFULL = EDGES_PER_SUB // CHUNK
TAIL = EDGES_PER_SUB - NFULL * CHUNK


def _spmm_one(src_hbm, dst_hbm, x_hbm, zeros_hbm, y_hbm,
              ysh, srci, dsti, srct, dstt, rows_v, semg, sems, semi):
    sid = lax.axis_index("s")
    r0 = pl.multiple_of(sid * ROWS_MAIN, 8)

    @pl.when(sid < 15)
    def _():
        pltpu.sync_copy(zeros_hbm.at[pl.ds(r0, ROWS_MAIN)],
                        ysh.at[pl.ds(r0, ROWS_MAIN)])

    @pl.when(sid == 15)
    def _():
        pltpu.sync_copy(zeros_hbm.at[pl.ds(15 * ROWS_MAIN, ROWS_LAST)],
                        ysh.at[pl.ds(15 * ROWS_MAIN, ROWS_LAST)])

    plsc.subcore_barrier()

    e0 = sid * EDGES_PER_SUB

    pltpu.sync_copy(src_hbm.at[pl.ds(pl.multiple_of(e0, 8), CHUNK)],
                    srci.at[0])
    pltpu.sync_copy(dst_hbm.at[pl.ds(pl.multiple_of(e0, 8), CHUNK)],
                    dsti.at[0])
    pltpu.async_copy(x_hbm.at[pl.ds(0, CHUNK)], rows_v.at[1], sems)
    pltpu.async_copy(x_hbm.at[pl.ds(0, CHUNK)], rows_v.at[1], sems)
    pltpu.async_copy(x_hbm.at[pl.ds(0, CHUNK)], rows_v.at[2], sems)

    def body(g, pb):
        ib = lax.bitwise_and(g, 3)
        ib1 = lax.bitwise_and(g + 1, 3)
        gn = lax.min(g + 1, NFULL - 1)
        base = pl.multiple_of(e0 + gn * CHUNK, 8)
        pltpu.make_async_copy(x_hbm.at[pl.ds(0, CHUNK)],
                              rows_v.at[pb], sems).wait()
        pltpu.async_copy(src_hbm.at[pl.ds(base, CHUNK)], srci.at[ib1], semi)
        pltpu.async_copy(dst_hbm.at[pl.ds(base, CHUNK)], dsti.at[ib1], semi)
        pltpu.async_copy(x_hbm.at[srci.at[ib]], rows_v.at[pb], semg)
        pltpu.make_async_copy(x_hbm.at[pl.ds(0, CHUNK)],
                              rows_v.at[pb], semg).wait()
        pltpu.async_copy(rows_v.at[pb], ysh.at[dsti.at[ib]], sems, add=True)
        pltpu.make_async_copy(src_hbm.at[pl.ds(0, CHUNK)],
                              srci.at[ib1], semi).wait()
        pltpu.make_async_copy(src_hbm.at[pl.ds(0, CHUNK)],
                              dsti.at[ib1], semi).wait()
        return lax.select(pb >= 2, 0, pb + 1)

    lax.fori_loop(0, NFULL, body, 0)

    for _ in range(3):
        pltpu.make_async_copy(x_hbm.at[pl.ds(0, CHUNK)],
                              rows_v.at[0], sems).wait()

    tb = pl.multiple_of(e0 + NFULL * CHUNK, 8)
    pltpu.sync_copy(src_hbm.at[pl.ds(tb, TAIL)], srct)
    pltpu.sync_copy(dst_hbm.at[pl.ds(tb, TAIL)], dstt)
    pltpu.async_copy(x_hbm.at[srct], rows_v.at[0, pl.ds(0, TAIL)],
                     semg).wait()
    pltpu.sync_copy(rows_v.at[0, pl.ds(0, TAIL)], ysh.at[dstt], add=True)

    plsc.subcore_barrier()

    @pl.when(sid < 15)
    def _():
        pltpu.sync_copy(ysh.at[pl.ds(r0, ROWS_MAIN)],
                        y_hbm.at[pl.ds(r0, ROWS_MAIN)])

    @pl.when(sid == 15)
    def _():
        pltpu.sync_copy(ysh.at[pl.ds(15 * ROWS_MAIN, ROWS_LAST)],
                        y_hbm.at[pl.ds(15 * ROWS_MAIN, ROWS_LAST)])


def _spmm_pair_body(src_a, dst_a, x_a, src_b, dst_b, x_b, zeros_hbm,
                    y_a, y_b,
                    ysh, srci, dsti, srct, dstt, rows_v, semg, sems, semi):
    cid = lax.axis_index("c")

    @pl.when(cid == 0)
    def _():
        _spmm_one(src_a, dst_a, x_a, zeros_hbm, y_a,
                  ysh, srci, dsti, srct, dstt, rows_v, semg, sems, semi)

    @pl.when(cid == 1)
    def _():
        _spmm_one(src_b, dst_b, x_b, zeros_hbm, y_b,
                  ysh, srci, dsti, srct, dstt, rows_v, semg, sems, semi)


@jax.jit
def _spmm_pair(src_a, dst_a, x_a, src_b, dst_b, x_b, zeros):
    mesh = plsc.VectorSubcoreMesh(core_axis_name="c", subcore_axis_name="s")
    f = pl.kernel(
        _spmm_pair_body,
        out_type=[jax.ShapeDtypeStruct((N, F), jnp.float32),
                  jax.ShapeDtypeStruct((N, F), jnp.float32)],
        mesh=mesh,
        scratch_types=[
            pltpu.VMEM_SHARED((N, F), jnp.float32),
            pltpu.VMEM((4, CHUNK), jnp.int32),
            pltpu.VMEM((4, CHUNK), jnp.int32),
            pltpu.VMEM((TAIL,), jnp.int32),
            pltpu.VMEM((TAIL,), jnp.int32),
            pltpu.VMEM((3, CHUNK, F), jnp.float32),
            pltpu.SemaphoreType.DMA,
            pltpu.SemaphoreType.DMA,
            pltpu.SemaphoreType.DMA,
        ],
    )
    return f(src_a, dst_a, x_a, src_b, dst_b, x_b, zeros)


_RB = 2000
_NB = N // _RB


def _leaky(x):
    return jnp.where(x >= 0, x, ALPHA * x)


def _mm(x, w):
    return jnp.dot(x, w, preferred_element_type=jnp.float32)


def _stage2_kernel(a1, a2, w1, b1, w2, b2, o1, o2):
    o1[...] = _leaky(_mm(a1[...], w1[...]) + b1[...])
    o2[...] = _leaky(_mm(a2[...], w2[...]) + b2[...])


@jax.jit
def _stage2(agg1, agg2, W1, b1, W2, b2):
    row = pl.BlockSpec((_RB, F), lambda i: (i, 0))
    wsp = pl.BlockSpec((F, F), lambda i: (0, 0))
    bsp = pl.BlockSpec((1, F), lambda i: (0, 0))
    return pl.pallas_call(
        _stage2_kernel,
        grid=(_NB,),
        in_specs=[row, row, wsp, bsp, wsp, bsp],
        out_specs=[row, row],
        out_shape=[jax.ShapeDtypeStruct((N, F), jnp.float32)] * 2,
    )(agg1, agg2, W1, b1.reshape(1, F), W2, b2.reshape(1, F))


def _stage4_kernel(a3, a4, uf, vf,
                   w3m, b3m, w3s, b3s, w4m, b4m, w4s, b4s,
                   wuma, wumb, bum, wusa, wusb, bus,
                   wima, wimb, bim, wisa, wisb, bis,
                   mu, lu, mi, li):
    x3, x4 = a3[...], a4[...]
    u, v = uf[...], vf[...]
    g3m = _leaky(_mm(x3, w3m[...]) + b3m[...])
    mu[...] = _mm(g3m, wuma[...]) + _mm(u, wumb[...]) + bum[...]
    g3s = _leaky(_mm(x3, w3s[...]) + b3s[...])
    lu[...] = _mm(g3s, wusa[...]) + _mm(u, wusb[...]) + bus[...]
    g4m = _leaky(_mm(x4, w4m[...]) + b4m[...])
    mi[...] = _mm(g4m, wima[...]) + _mm(v, wimb[...]) + bim[...]
    g4s = _leaky(_mm(x4, w4s[...]) + b4s[...])
    li[...] = _mm(g4s, wisa[...]) + _mm(v, wisb[...]) + bis[...]


@jax.jit
def _stage4(agg3, agg4, ufea, vfea,
            W3m, b3m, W3s, b3s, W4m, b4m, W4s, b4s,
            Wum, bum, Wus, bus, Wim, bim, Wis, bis):
    row = pl.BlockSpec((_RB, F), lambda i: (i, 0))
    wsp = pl.BlockSpec((F, F), lambda i: (0, 0))
    bsp = pl.BlockSpec((1, F), lambda i: (0, 0))
    args = [
        agg3, agg4, ufea, vfea,
        W3m, b3m.reshape(1, F), W3s, b3s.reshape(1, F),
        W4m, b4m.reshape(1, F), W4s, b4s.reshape(1, F),
        Wum[:F], Wum[F:], bum.reshape(1, F),
        Wus[:F], Wus[F:], bus.reshape(1, F),
        Wim[:F], Wim[F:], bim.reshape(1, F),
        Wis[:F], Wis[F:], bis.reshape(1, F),
    ]
    specs = [row, row, row, row] + [wsp, bsp] * 4 + [wsp, wsp, bsp] * 4
    return pl.pallas_call(
        _stage4_kernel,
        grid=(_NB,),
        in_specs=specs,
        out_specs=[row] * 4,
        out_shape=[jax.ShapeDtypeStruct((N, F), jnp.float32)] * 4,
    )(*args)


def kernel(ufea, vfea, UV_adj, VU_adj, W_gc1, b_gc1, W_gc2, b_gc2,
           W_gc3m, b_gc3m, W_gc3s, b_gc3s, W_gc4m, b_gc4m, W_gc4s, b_gc4s,
           W_uum, b_uum, W_uus, b_uus, W_ium, b_ium, W_ius, b_ius):
    zeros = jnp.zeros((N, F), jnp.float32)

    uv_s, uv_d = UV_adj[1], UV_adj[0]
    vu_s, vu_d = VU_adj[1], VU_adj[0]

    agg1, agg2 = _spmm_pair(vu_s, vu_d, ufea, uv_s, uv_d, vfea, zeros)
    ho1, ho2 = _stage2(agg1, agg2, W_gc1, b_gc1, W_gc2, b_gc2)
    agg3, agg4 = _spmm_pair(uv_s, uv_d, ho1, vu_s, vu_d, ho2, zeros)
    mean_u, logstd_u, mean_i, logstd_i = _stage4(
        agg3, agg4, ufea, vfea,
        W_gc3m, b_gc3m, W_gc3s, b_gc3s, W_gc4m, b_gc4m, W_gc4s, b_gc4s,
        W_uum, b_uum, W_uus, b_uus, W_ium, b_ium, W_ius, b_ius)
    return (mean_u, mean_i, mean_u, mean_i, logstd_u, logstd_i)


# speedup vs baseline: 1.0407x; 1.0407x over previous
import functools

import jax
import jax.numpy as jnp
from jax import lax
from jax.experimental import pallas as pl
from jax.experimental.pallas import tpu as pltpu
from jax.experimental.pallas import tpu_sc as plsc

N = 10000
F = 128
E = 320000
ALPHA = 0.3

NSUB = 16
ROWS_MAIN = 632
ROWS_LAST = N - 15 * ROWS_MAIN
CHUNK = 128
NCHUNK = E // CHUNK
NCH_BASE = NCHUNK // NSUB


def _spmm_one(ed_hbm, x_hbm, zeros_hbm, y_hbm,
              ysh, idxv, rows_v, semg, sems, semi):
    sid = lax.axis_index("s")
    r0 = pl.multiple_of(sid * ROWS_MAIN, 8)

    @pl.when(sid < 15)
    def _():
        pltpu.sync_copy(zeros_hbm.at[pl.ds(r0, ROWS_MAIN)],
                        ysh.at[pl.ds(r0, ROWS_MAIN)])

    @pl.when(sid == 15)
    def _():
        pltpu.sync_copy(zeros_hbm.at[pl.ds(15 * ROWS_MAIN, ROWS_LAST)],
                        ysh.at[pl.ds(15 * ROWS_MAIN, ROWS_LAST)])

    plsc.subcore_barrier()

    c0 = sid * NCH_BASE + lax.min(sid, 4)
    nch = NCH_BASE + lax.select(sid < 4, 1, 0)
    e0 = c0 * CHUNK

    pltpu.sync_copy(ed_hbm.at[:, pl.ds(pl.multiple_of(e0, CHUNK), CHUNK)],
                    idxv.at[0])
    pltpu.async_copy(x_hbm.at[pl.ds(0, CHUNK)], rows_v.at[1], sems)
    pltpu.async_copy(x_hbm.at[pl.ds(0, CHUNK)], rows_v.at[1], sems)

    def body(g, carry):
        pb = lax.bitwise_and(g, 1)
        ib = lax.bitwise_and(g, 3)
        ib1 = lax.bitwise_and(g + 1, 3)
        gn = lax.min(g + 1, nch - 1)
        base = pl.multiple_of(e0 + gn * CHUNK, CHUNK)
        pltpu.make_async_copy(x_hbm.at[pl.ds(0, CHUNK)],
                              rows_v.at[pb], sems).wait()
        pltpu.async_copy(ed_hbm.at[:, pl.ds(base, CHUNK)], idxv.at[ib1],
                         semi)
        pltpu.async_copy(x_hbm.at[idxv.at[ib, 1]], rows_v.at[pb], semg)
        pltpu.make_async_copy(x_hbm.at[pl.ds(0, CHUNK)],
                              rows_v.at[pb], semg).wait()
        pltpu.async_copy(rows_v.at[pb], ysh.at[idxv.at[ib, 0]], sems,
                         add=True)
        pltpu.make_async_copy(ed_hbm.at[:, pl.ds(0, CHUNK)],
                              idxv.at[ib1], semi).wait()
        return carry

    lax.fori_loop(0, nch, body, 0)

    for _ in range(2):
        pltpu.make_async_copy(x_hbm.at[pl.ds(0, CHUNK)],
                              rows_v.at[0], sems).wait()

    plsc.subcore_barrier()

    @pl.when(sid < 15)
    def _():
        pltpu.sync_copy(ysh.at[pl.ds(r0, ROWS_MAIN)],
                        y_hbm.at[pl.ds(r0, ROWS_MAIN)])

    @pl.when(sid == 15)
    def _():
        pltpu.sync_copy(ysh.at[pl.ds(15 * ROWS_MAIN, ROWS_LAST)],
                        y_hbm.at[pl.ds(15 * ROWS_MAIN, ROWS_LAST)])


def _spmm_pair_body(ed_a, x_a, ed_b, x_b, zeros_hbm,
                    y_a, y_b,
                    ysh, idxv, rows_v, semg, sems, semi):
    cid = lax.axis_index("c")

    @pl.when(cid == 0)
    def _():
        _spmm_one(ed_a, x_a, zeros_hbm, y_a,
                  ysh, idxv, rows_v, semg, sems, semi)

    @pl.when(cid == 1)
    def _():
        _spmm_one(ed_b, x_b, zeros_hbm, y_b,
                  ysh, idxv, rows_v, semg, sems, semi)


@jax.jit
def _spmm_pair(ed_a, x_a, ed_b, x_b, zeros):
    mesh = plsc.VectorSubcoreMesh(core_axis_name="c", subcore_axis_name="s")
    f = pl.kernel(
        _spmm_pair_body,
        out_type=[jax.ShapeDtypeStruct((N, F), jnp.float32),
                  jax.ShapeDtypeStruct((N, F), jnp.float32)],
        mesh=mesh,
        scratch_types=[
            pltpu.VMEM_SHARED((N, F), jnp.float32),
            pltpu.VMEM((4, 2, CHUNK), jnp.int32),
            pltpu.VMEM((2, CHUNK, F), jnp.float32),
            pltpu.SemaphoreType.DMA,
            pltpu.SemaphoreType.DMA,
            pltpu.SemaphoreType.DMA,
        ],
    )
    return f(ed_a, x_a, ed_b, x_b, zeros)


_RB = 2000
_NB = N // _RB


def _leaky(x):
    return jnp.where(x >= 0, x, ALPHA * x)


def _mm(x, w):
    return jnp.dot(x, w, preferred_element_type=jnp.float32)


def _stage2_kernel(a1, a2, w1, b1, w2, b2, o1, o2):
    o1[...] = _leaky(_mm(a1[...], w1[...]) + b1[...])
    o2[...] = _leaky(_mm(a2[...], w2[...]) + b2[...])


@jax.jit
def _stage2(agg1, agg2, W1, b1, W2, b2):
    row = pl.BlockSpec((_RB, F), lambda i: (i, 0))
    wsp = pl.BlockSpec((F, F), lambda i: (0, 0))
    bsp = pl.BlockSpec((1, F), lambda i: (0, 0))
    return pl.pallas_call(
        _stage2_kernel,
        grid=(_NB,),
        in_specs=[row, row, wsp, bsp, wsp, bsp],
        out_specs=[row, row],
        out_shape=[jax.ShapeDtypeStruct((N, F), jnp.float32)] * 2,
    )(agg1, agg2, W1, b1.reshape(1, F), W2, b2.reshape(1, F))


def _stage4_kernel(a3, a4, uf, vf,
                   w3m, b3m, w3s, b3s, w4m, b4m, w4s, b4s,
                   wuma, wumb, bum, wusa, wusb, bus,
                   wima, wimb, bim, wisa, wisb, bis,
                   mu, lu, mi, li):
    x3, x4 = a3[...], a4[...]
    u, v = uf[...], vf[...]
    g3m = _leaky(_mm(x3, w3m[...]) + b3m[...])
    mu[...] = _mm(g3m, wuma[...]) + _mm(u, wumb[...]) + bum[...]
    g3s = _leaky(_mm(x3, w3s[...]) + b3s[...])
    lu[...] = _mm(g3s, wusa[...]) + _mm(u, wusb[...]) + bus[...]
    g4m = _leaky(_mm(x4, w4m[...]) + b4m[...])
    mi[...] = _mm(g4m, wima[...]) + _mm(v, wimb[...]) + bim[...]
    g4s = _leaky(_mm(x4, w4s[...]) + b4s[...])
    li[...] = _mm(g4s, wisa[...]) + _mm(v, wisb[...]) + bis[...]


@jax.jit
def _stage4(agg3, agg4, ufea, vfea,
            W3m, b3m, W3s, b3s, W4m, b4m, W4s, b4s,
            Wum, bum, Wus, bus, Wim, bim, Wis, bis):
    row = pl.BlockSpec((_RB, F), lambda i: (i, 0))
    wsp = pl.BlockSpec((F, F), lambda i: (0, 0))
    bsp = pl.BlockSpec((1, F), lambda i: (0, 0))
    args = [
        agg3, agg4, ufea, vfea,
        W3m, b3m.reshape(1, F), W3s, b3s.reshape(1, F),
        W4m, b4m.reshape(1, F), W4s, b4s.reshape(1, F),
        Wum[:F], Wum[F:], bum.reshape(1, F),
        Wus[:F], Wus[F:], bus.reshape(1, F),
        Wim[:F], Wim[F:], bim.reshape(1, F),
        Wis[:F], Wis[F:], bis.reshape(1, F),
    ]
    specs = [row, row, row, row] + [wsp, bsp] * 4 + [wsp, wsp, bsp] * 4
    return pl.pallas_call(
        _stage4_kernel,
        grid=(_NB,),
        in_specs=specs,
        out_specs=[row] * 4,
        out_shape=[jax.ShapeDtypeStruct((N, F), jnp.float32)] * 4,
    )(*args)


def kernel(ufea, vfea, UV_adj, VU_adj, W_gc1, b_gc1, W_gc2, b_gc2,
           W_gc3m, b_gc3m, W_gc3s, b_gc3s, W_gc4m, b_gc4m, W_gc4s, b_gc4s,
           W_uum, b_uum, W_uus, b_uus, W_ium, b_ium, W_ius, b_ius):
    zeros = jnp.zeros((N, F), jnp.float32)

    agg1, agg2 = _spmm_pair(VU_adj, ufea, UV_adj, vfea, zeros)
    ho1, ho2 = _stage2(agg1, agg2, W_gc1, b_gc1, W_gc2, b_gc2)
    agg3, agg4 = _spmm_pair(UV_adj, ho1, VU_adj, ho2, zeros)
    mean_u, logstd_u, mean_i, logstd_i = _stage4(
        agg3, agg4, ufea, vfea,
        W_gc3m, b_gc3m, W_gc3s, b_gc3s, W_gc4m, b_gc4m, W_gc4s, b_gc4s,
        W_uum, b_uum, W_uus, b_uus, W_ium, b_ium, W_ius, b_ius)
    return (mean_u, mean_i, mean_u, mean_i, logstd_u, logstd_i)
